# baseline (speedup 1.0000x reference)
import functools

import jax
import jax.numpy as jnp
import numpy as np
from jax import lax
from jax.experimental import pallas as pl
from jax.experimental.pallas import tpu as pltpu
from jax.experimental.pallas import tpu_sc as plsc


def _sc_gather_sum(x_flat, emb_bf, *, interpret=False):
    S = 32
    total_idx = x_flat.shape[0]
    B = total_idx // S
    D = emb_bf.shape[1]
    NC, NS = 2, 16
    NW = NC * NS
    BPW = B // NW
    CH = 4
    ROWS = CH * S
    NBUF = 2
    NCHUNK = BPW // CH
    NG = D // 32
    V = emb_bf.shape[0]
    VPS = V // NS

    mesh = plsc.VectorSubcoreMesh(core_axis_name="c", subcore_axis_name="s")

    @functools.partial(
        pl.kernel,
        out_type=jax.ShapeDtypeStruct((B, D), jnp.float32),
        mesh=mesh,
        scratch_types=[
            pltpu.VMEM((BPW * S,), jnp.int32),
            pltpu.VMEM((NBUF, ROWS, D), jnp.bfloat16),
            pltpu.VMEM((BPW, D), jnp.float32),
            pltpu.VMEM_SHARED((V, D), jnp.bfloat16),
            pltpu.SemaphoreType.DMA,
            pltpu.SemaphoreType.DMA,
        ],
        compiler_params=pltpu.CompilerParams(use_tc_tiling_on_sc=False),
        interpret=interpret,
    )
    def gather_kernel(x_hbm, emb_hbm, out_hbm, x_v, rows_v, out_v, tbl_sh,
                      sem0, sem1):
        wid = lax.axis_index("s") * NC + lax.axis_index("c")
        sid = lax.axis_index("s")
        base = wid * BPW
        sems = (sem0, sem1)

        pltpu.sync_copy(emb_hbm.at[pl.ds(sid * VPS, VPS)],
                        tbl_sh.at[pl.ds(sid * VPS, VPS)])

        pltpu.sync_copy(x_hbm.at[pl.ds(base * S, BPW * S)], x_v)
        plsc.subcore_barrier()

        def issue(c, b):
            pltpu.async_copy(
                tbl_sh.at[x_v.at[pl.ds(c * ROWS, ROWS)]],
                rows_v.at[b],
                sems[b],
            )

        def wait(b):
            pltpu.make_async_copy(
                tbl_sh.at[x_v.at[pl.ds(0, ROWS)]],
                rows_v.at[b],
                sems[b],
            ).wait()

        for b in range(NBUF):
            issue(b, b)

        def outer(g, _):
            for b in range(NBUF):
                c = g * NBUF + b
                wait(b)
                for e in range(CH):
                    r0 = e * S
                    for dg in range(NG):
                        sl = pl.ds(dg * 32, 32)
                        vals = [rows_v[b, r0 + j, sl] for j in range(29)]
                        for _lvl in range(2):
                            nxt_vals = [vals[i] + vals[i + 1]
                                        for i in range(0, len(vals) - 1, 2)]
                            if len(vals) % 2:
                                nxt_vals.append(vals[-1])
                            vals = nxt_vals
                        acc0 = vals[0].astype(jnp.float32)
                        acc1 = vals[1].astype(jnp.float32)
                        for i in range(2, len(vals) - 1, 2):
                            acc0 = acc0 + vals[i].astype(jnp.float32)
                            acc1 = acc1 + vals[i + 1].astype(jnp.float32)
                        if len(vals) % 2:
                            acc0 = acc0 + vals[-1].astype(jnp.float32)
                        out_v[c * CH + e, sl] = acc0 + acc1
                nxt = c + NBUF

                @pl.when(nxt < NCHUNK)
                def _():
                    issue(nxt, b)
            return _

        lax.fori_loop(0, NCHUNK // NBUF, outer, None)
        pltpu.sync_copy(out_v, out_hbm.at[pl.ds(base, BPW)])

    return gather_kernel(x_flat, emb_bf)


def _mlp_body(pooled_ref, w2_ref, b2_ref, w3_ref, b3_ref, w4_ref, out_ref):
    h = jnp.maximum(pooled_ref[...], 0.0)
    a = lax.dot_general(h, w2_ref[...], (((1,), (1,)), ((), ())),
                        preferred_element_type=jnp.float32)
    a = a + b2_ref[...][None, :]
    h = jnp.concatenate((a, -a), axis=-1)
    h = jnp.maximum(h, 0.0)
    a = lax.dot_general(h, w3_ref[...], (((1,), (1,)), ((), ())),
                        preferred_element_type=jnp.float32)
    a = a + b3_ref[...][None, :]
    h = jnp.concatenate((a, -a), axis=-1)
    h = jnp.maximum(h, 0.0)
    out_ref[...] = lax.dot_general(h, w4_ref[...], (((1,), (1,)), ((), ())),
                                   preferred_element_type=jnp.float32)


def _tc_mlp(pooled, W2, b2, W3, b3, W4, *, interpret=False):
    B, Wp = pooled.shape
    BB = 2048
    grid = (B // BB,)
    return pl.pallas_call(
        _mlp_body,
        grid=grid,
        in_specs=[
            pl.BlockSpec((BB, Wp), lambda i: (i, 0)),
            pl.BlockSpec(W2.shape, lambda i: (0, 0)),
            pl.BlockSpec(b2.shape, lambda i: (0,)),
            pl.BlockSpec(W3.shape, lambda i: (0, 0)),
            pl.BlockSpec(b3.shape, lambda i: (0,)),
            pl.BlockSpec(W4.shape, lambda i: (0, 0)),
        ],
        out_specs=pl.BlockSpec((BB, 1), lambda i: (i, 0)),
        out_shape=jax.ShapeDtypeStruct((B, 1), jnp.float32),
        interpret=interpret,
    )(pooled, W2, b2, W3, b3, W4)


def kernel(x, emb, W2, b2, W3, b3, W4):
    x_flat = x.astype(jnp.int32).reshape(-1)
    pooled = _sc_gather_sum(x_flat, emb.astype(jnp.bfloat16))
    return _tc_mlp(pooled, W2, b2, W3, b3, W4)


# speedup vs baseline: 1.5451x; 1.5451x over previous
import functools

import jax
import jax.numpy as jnp
import numpy as np
from jax import lax
from jax.experimental import pallas as pl
from jax.experimental.pallas import tpu as pltpu
from jax.experimental.pallas import tpu_sc as plsc


def _sc_gather_sum(x_flat, emb_bf, *, interpret=False):
    S = 32
    total_idx = x_flat.shape[0]
    B = total_idx // S
    D = emb_bf.shape[1]
    NC, NS = 2, 16
    NW = NC * NS
    BPW = B // NW
    CH = 4
    ROWS = CH * S
    NBUF = 2
    NCHUNK = BPW // CH
    NG = D // 32
    V = emb_bf.shape[0]
    VPS = V // NS

    mesh = plsc.VectorSubcoreMesh(core_axis_name="c", subcore_axis_name="s")

    @functools.partial(
        pl.kernel,
        out_type=jax.ShapeDtypeStruct((B, D), jnp.float32),
        mesh=mesh,
        scratch_types=[
            pltpu.VMEM((BPW * S,), jnp.int32),
            pltpu.VMEM((NBUF, ROWS, D), jnp.bfloat16),
            pltpu.VMEM((BPW, D), jnp.float32),
            pltpu.VMEM_SHARED((V, D), jnp.bfloat16),
            pltpu.SemaphoreType.DMA,
            pltpu.SemaphoreType.DMA,
        ],
        compiler_params=pltpu.CompilerParams(use_tc_tiling_on_sc=False),
        interpret=interpret,
    )
    def gather_kernel(x_hbm, emb_hbm, out_hbm, x_v, rows_v, out_v, tbl_sh,
                      sem0, sem1):
        wid = lax.axis_index("s") * NC + lax.axis_index("c")
        sid = lax.axis_index("s")
        base = wid * BPW
        sems = (sem0, sem1)

        pltpu.sync_copy(emb_hbm.at[pl.ds(sid * VPS, VPS)],
                        tbl_sh.at[pl.ds(sid * VPS, VPS)])

        pltpu.sync_copy(x_hbm.at[pl.ds(base * S, BPW * S)], x_v)
        plsc.subcore_barrier()

        def issue(c, b):
            pltpu.async_copy(
                tbl_sh.at[x_v.at[pl.ds(c * ROWS, ROWS)]],
                rows_v.at[b],
                sems[b],
            )

        def wait(b):
            pltpu.make_async_copy(
                tbl_sh.at[x_v.at[pl.ds(0, ROWS)]],
                rows_v.at[b],
                sems[b],
            ).wait()

        for b in range(NBUF):
            issue(b, b)

        def outer(g, _):
            for b in range(NBUF):
                c = g * NBUF + b
                wait(b)
                for e in range(CH):
                    r0 = e * S
                    for dg in range(NG):
                        sl = pl.ds(dg * 32, 32)
                        vals = [rows_v[b, r0 + j, sl] for j in range(29)]
                        for _lvl in range(3):
                            nxt_vals = [vals[i] + vals[i + 1]
                                        for i in range(0, len(vals) - 1, 2)]
                            if len(vals) % 2:
                                nxt_vals.append(vals[-1])
                            vals = nxt_vals
                        vals = [v.astype(jnp.float32) for v in vals]
                        while len(vals) > 1:
                            nxt_vals = [vals[i] + vals[i + 1]
                                        for i in range(0, len(vals) - 1, 2)]
                            if len(vals) % 2:
                                nxt_vals.append(vals[-1])
                            vals = nxt_vals
                        out_v[c * CH + e, sl] = vals[0]
                nxt = c + NBUF

                @pl.when(nxt < NCHUNK)
                def _():
                    issue(nxt, b)
            return _

        lax.fori_loop(0, NCHUNK // NBUF, outer, None)
        pltpu.sync_copy(out_v, out_hbm.at[pl.ds(base, BPW)])

    return gather_kernel(x_flat, emb_bf)


def _mlp_body(pooled_ref, w2_ref, b2_ref, w3_ref, b3_ref, w4_ref, out_ref):
    h = jnp.maximum(pooled_ref[...], 0.0)
    a = lax.dot_general(h, w2_ref[...], (((1,), (1,)), ((), ())),
                        preferred_element_type=jnp.float32)
    a = a + b2_ref[...][None, :]
    h = jnp.concatenate((a, -a), axis=-1)
    h = jnp.maximum(h, 0.0)
    a = lax.dot_general(h, w3_ref[...], (((1,), (1,)), ((), ())),
                        preferred_element_type=jnp.float32)
    a = a + b3_ref[...][None, :]
    h = jnp.concatenate((a, -a), axis=-1)
    h = jnp.maximum(h, 0.0)
    out_ref[...] = lax.dot_general(h, w4_ref[...], (((1,), (1,)), ((), ())),
                                   preferred_element_type=jnp.float32)


def _tc_mlp(pooled, W2, b2, W3, b3, W4, *, interpret=False):
    B, Wp = pooled.shape
    BB = 4096
    grid = (B // BB,)
    return pl.pallas_call(
        _mlp_body,
        grid=grid,
        in_specs=[
            pl.BlockSpec((BB, Wp), lambda i: (i, 0)),
            pl.BlockSpec(W2.shape, lambda i: (0, 0)),
            pl.BlockSpec(b2.shape, lambda i: (0,)),
            pl.BlockSpec(W3.shape, lambda i: (0, 0)),
            pl.BlockSpec(b3.shape, lambda i: (0,)),
            pl.BlockSpec(W4.shape, lambda i: (0, 0)),
        ],
        out_specs=pl.BlockSpec((BB, 1), lambda i: (i, 0)),
        out_shape=jax.ShapeDtypeStruct((B, 1), jnp.float32),
        interpret=interpret,
    )(pooled, W2, b2, W3, b3, W4)


def kernel(x, emb, W2, b2, W3, b3, W4):
    x_flat = x.astype(jnp.int32).reshape(-1)
    pooled = _sc_gather_sum(x_flat, emb.astype(jnp.bfloat16))
    return _tc_mlp(pooled, W2, b2, W3, b3, W4)
